# baseline (speedup 1.0000x reference)
import functools

import jax
import jax.numpy as jnp
from jax import lax
from jax.experimental import pallas as pl
from jax.experimental.pallas import tpu as pltpu
from jax.experimental.pallas import tpu_sc as plsc

_N = 4096
_NCOMBO = 4
_CHUNK = 512
_GROUPS = _CHUNK // 16
_GBLK = 8


def _newton_sqrt(x):
    i = lax.bitcast_convert_type(x, jnp.int32)
    y = lax.bitcast_convert_type(
        (i >> 1) + jnp.int32(0x1FBD3F7D), jnp.float32)
    half = jnp.float32(0.5)
    y = half * (y + x / y)
    y = half * (y + x / y)
    y = half * (y + x / y)
    return y


def _chamfer_sc(q_hbm, out_hbm, qx, qy, qz, cx, cy, cz, ch, accv):
    nc = 2
    wid = lax.axis_index("s") * nc + lax.axis_index("c")
    combo = wid // 8
    chunk = wid % 8
    ccombo = combo ^ 1
    qbase = chunk * _CHUNK

    qrow = combo * (3 * _N)
    crow = ccombo * (3 * _N)
    pltpu.sync_copy(q_hbm.at[pl.ds(qrow, _N)], qx)
    pltpu.sync_copy(q_hbm.at[pl.ds(qrow + _N, _N)], qy)
    pltpu.sync_copy(q_hbm.at[pl.ds(qrow + 2 * _N, _N)], qz)
    pltpu.sync_copy(q_hbm.at[pl.ds(crow, _N)], cx)
    pltpu.sync_copy(q_hbm.at[pl.ds(crow + _N, _N)], cy)
    pltpu.sync_copy(q_hbm.at[pl.ds(crow + 2 * _N, _N)], cz)

    def hbody(jb, _):
        base = jb * 16
        cxv = cx[pl.ds(base, 16)]
        cyv = cy[pl.ds(base, 16)]
        czv = cz[pl.ds(base, 16)]
        ch[pl.ds(base, 16)] = (cxv * cxv + cyv * cyv + czv * czv) * 0.5
        return 0

    lax.fori_loop(0, _N // 16, hbody, 0)

    acc = jnp.zeros((16,), jnp.float32)
    neg = jnp.full((16,), -3.0e38, jnp.float32)

    for blk in range(_GROUPS // _GBLK):
        qvs = []
        for g in range(_GBLK):
            off = qbase + (blk * _GBLK + g) * 16
            qvs.append((qx[pl.ds(off, 16)],
                        qy[pl.ds(off, 16)],
                        qz[pl.ds(off, 16)]))

        def body(jb, scores, qvs=qvs):
            base = jb * 16
            cxv = cx[pl.ds(base, 16)]
            cyv = cy[pl.ds(base, 16)]
            czv = cz[pl.ds(base, 16)]
            chv = ch[pl.ds(base, 16)]
            out = list(scores)
            for lane in range(16):
                bx = cxv[lane]
                by = cyv[lane]
                bz = czv[lane]
                bh = chv[lane]
                for g in range(_GBLK):
                    s = qvs[g][0] * bx + (qvs[g][1] * by
                                          + (qvs[g][2] * bz - bh))
                    out[g] = jnp.maximum(out[g], s)
            return tuple(out)

        scores = lax.fori_loop(0, _N // 16, body, tuple([neg] * _GBLK))
        for g in range(_GBLK):
            q2 = (qvs[g][0] * qvs[g][0] + qvs[g][1] * qvs[g][1]
                  + qvs[g][2] * qvs[g][2])
            d2 = jnp.maximum(q2 - 2.0 * scores[g], 0.0)
            acc = acc + _newton_sqrt(d2)

    accv[...] = acc
    pltpu.sync_copy(accv, out_hbm.at[pl.ds(wid * 16, 16)])


def kernel(pc1, pc2):
    b = pc1.shape[0]
    n = pc1.shape[1]
    q = jnp.stack([pc1[0].T, pc2[0].T, pc1[1].T, pc2[1].T]).reshape(-1)

    mesh = plsc.VectorSubcoreMesh(core_axis_name="c", subcore_axis_name="s")
    run = functools.partial(
        pl.kernel,
        mesh=mesh,
        out_type=jax.ShapeDtypeStruct((32 * 16,), jnp.float32),
        scratch_types=[pltpu.VMEM((n,), jnp.float32)] * 7
        + [pltpu.VMEM((16,), jnp.float32)],
    )(_chamfer_sc)
    partials = run(q)
    return jnp.sum(partials) / jnp.float32(2 * n * b)


# speedup vs baseline: 11.7419x; 11.7419x over previous
import functools

import jax
import jax.numpy as jnp
from jax import lax
from jax.experimental import pallas as pl
from jax.experimental.pallas import tpu as pltpu
from jax.experimental.pallas import tpu_sc as plsc

_N = 4096
_NCOMBO = 4
_SC_Q = 1024
_CHUNK = _SC_Q // 8
_GBLK = 8
_TC_TILE = 256


def _newton_sqrt(x):
    i = lax.bitcast_convert_type(x, jnp.int32)
    y = lax.bitcast_convert_type(
        (i >> 1) + jnp.int32(0x1FBD3F7D), jnp.float32)
    half = jnp.float32(0.5)
    y = half * (y + x / y)
    y = half * (y + x / y)
    y = half * (y + x / y)
    return y


def _chamfer_sc(q_hbm, out_hbm, qx, qy, qz, cx, cy, cz, accv):
    nc = 2
    wid = lax.axis_index("s") * nc + lax.axis_index("c")
    combo = wid // 8
    chunk = wid % 8
    ccombo = combo ^ 1
    qbase = chunk * _CHUNK

    qrow = combo * (3 * _N)
    crow = ccombo * (3 * _N)
    pltpu.sync_copy(q_hbm.at[pl.ds(qrow + qbase, _CHUNK)], qx)
    pltpu.sync_copy(q_hbm.at[pl.ds(qrow + _N + qbase, _CHUNK)], qy)
    pltpu.sync_copy(q_hbm.at[pl.ds(qrow + 2 * _N + qbase, _CHUNK)], qz)
    pltpu.sync_copy(q_hbm.at[pl.ds(crow, _N)], cx)
    pltpu.sync_copy(q_hbm.at[pl.ds(crow + _N, _N)], cy)
    pltpu.sync_copy(q_hbm.at[pl.ds(crow + 2 * _N, _N)], cz)

    acc = jnp.zeros((16,), jnp.float32)
    big = jnp.full((16,), 3.0e38, jnp.float32)

    for blk in range(_CHUNK // 16 // _GBLK):
        qvs = []
        for g in range(_GBLK):
            off = (blk * _GBLK + g) * 16
            qvs.append((qx[pl.ds(off, 16)],
                        qy[pl.ds(off, 16)],
                        qz[pl.ds(off, 16)]))

        def body(jb, dmins, qvs=qvs):
            base = jb * 16
            cxv = cx[pl.ds(base, 16)]
            cyv = cy[pl.ds(base, 16)]
            czv = cz[pl.ds(base, 16)]
            out = list(dmins)
            for lane in range(16):
                bx = cxv[lane]
                by = cyv[lane]
                bz = czv[lane]
                for g in range(_GBLK):
                    dx = qvs[g][0] - bx
                    dy = qvs[g][1] - by
                    dz = qvs[g][2] - bz
                    d2 = dx * dx + dy * dy + dz * dz
                    out[g] = jnp.minimum(out[g], d2)
            return tuple(out)

        dmins = lax.fori_loop(0, _N // 16, body, tuple([big] * _GBLK))
        for g in range(_GBLK):
            acc = acc + _newton_sqrt(dmins[g])

    accv[...] = acc
    pltpu.sync_copy(accv, out_hbm.at[pl.ds(wid * 16, 16)])


def _chamfer_tc(q_ref, c_ref, o_ref):
    q = q_ref[0]
    cx = c_ref[0, 0:1, :]
    cy = c_ref[0, 1:2, :]
    cz = c_ref[0, 2:3, :]
    dx = q[:, 0:1] - cx
    d2 = dx * dx
    dy = q[:, 1:2] - cy
    d2 = d2 + dy * dy
    dz = q[:, 2:3] - cz
    d2 = d2 + dz * dz
    m = jnp.min(d2, axis=1)
    o_ref[0, 0, pl.program_id(1)] = jnp.sum(jnp.sqrt(m))


def kernel(pc1, pc2):
    b = pc1.shape[0]
    n = pc1.shape[1]
    qsoa = jnp.stack([pc1[0].T, pc2[0].T, pc1[1].T, pc2[1].T])
    csoa = qsoa[jnp.array([1, 0, 3, 2])]
    qaos = jnp.stack([pc1[0], pc2[0], pc1[1], pc2[1]])

    mesh = plsc.VectorSubcoreMesh(core_axis_name="c", subcore_axis_name="s")
    sc_run = functools.partial(
        pl.kernel,
        mesh=mesh,
        out_type=jax.ShapeDtypeStruct((32 * 16,), jnp.float32),
        scratch_types=[pltpu.VMEM((_CHUNK,), jnp.float32)] * 3
        + [pltpu.VMEM((n,), jnp.float32)] * 3
        + [pltpu.VMEM((16,), jnp.float32)],
    )(_chamfer_sc)
    sc_part = sc_run(qsoa.reshape(-1))

    tc_q = qaos[:, _SC_Q:, :]
    ntiles = (n - _SC_Q) // _TC_TILE
    tc_part = pl.pallas_call(
        _chamfer_tc,
        grid=(_NCOMBO, ntiles),
        in_specs=[
            pl.BlockSpec((1, _TC_TILE, 3), lambda i, j: (i, j, 0)),
            pl.BlockSpec((1, 3, n), lambda i, j: (i, 0, 0)),
        ],
        out_specs=pl.BlockSpec(
            (1, 1, ntiles), lambda i, j: (i, 0, 0), memory_space=pltpu.SMEM),
        out_shape=jax.ShapeDtypeStruct((_NCOMBO, 1, ntiles), jnp.float32),
    )(tc_q, csoa)

    total = jnp.sum(sc_part) + jnp.sum(tc_part)
    return total / jnp.float32(2 * n * b)
